# Initial kernel scaffold; baseline (speedup 1.0000x reference)
import functools

import jax
import jax.numpy as jnp
from jax import lax
from jax.experimental import pallas as pl
from jax.experimental.pallas import tpu as pltpu
from jax.experimental.pallas import tpu_sc as plsc

_B, _C, _N = 2, 16, 8192
_DS = 4
_M = _N // _DS
_K = 32
_XYZN = 7
_EPS = 1e-5
_QT = 128
_ST = 2048
_MT = 256
_TOTAL = _B * _K * _M


def _knn_kern(pts_ref, q_ref, out_ref, d_ref):
    px = pts_ref[0, 0:1, :]
    py = pts_ref[0, 1:2, :]
    pz = pts_ref[0, 2:3, :]
    qx = q_ref[0, :, 0:1]
    qy = q_ref[0, :, 1:2]
    qz = q_ref[0, :, 2:3]
    d_ref[...] = (qx - px) ** 2 + (qy - py) ** 2 + (qz - pz) ** 2
    iota = lax.broadcasted_iota(jnp.int32, (_QT, _N), 1)

    def body(k, _):
        d = d_ref[...]
        mn = jnp.min(d, axis=1, keepdims=True)
        am = jnp.min(jnp.where(d == mn, iota, _N), axis=1)
        out_ref[0, pl.ds(k, 1), :] = am[None, :]
        d_ref[...] = jnp.where(iota == am[:, None], jnp.inf, d)
        return 0

    lax.fori_loop(0, _K, body, 0)


def _knn(pts, qT):
    return pl.pallas_call(
        _knn_kern,
        grid=(_B, _M // _QT),
        in_specs=[
            pl.BlockSpec((1, 3, _N), lambda b, t: (b, 0, 0)),
            pl.BlockSpec((1, _QT, 3), lambda b, t: (b, t, 0)),
        ],
        out_specs=pl.BlockSpec((1, _K, _QT), lambda b, t: (b, 0, t)),
        out_shape=jax.ShapeDtypeStruct((_B, _K, _M), jnp.int32),
        scratch_shapes=[pltpu.VMEM((_QT, _N), jnp.float32)],
    )(pts, qT)


_NW = 32
_CH = 128


def _gather_sc(table, flat_idx):
    per_w = _TOTAL // _NW
    n_ch = per_w // _CH
    info = plsc.get_sparse_core_info()
    nc = info.num_cores
    mesh = plsc.VectorSubcoreMesh(core_axis_name="c", subcore_axis_name="s")

    @functools.partial(
        pl.kernel,
        mesh=mesh,
        out_type=jax.ShapeDtypeStruct((_TOTAL, _C), jnp.float32),
        scratch_types=[
            pltpu.VMEM((_CH,), jnp.int32),
            pltpu.VMEM((_CH, _C), jnp.float32),
            pltpu.SemaphoreType.DMA,
        ],
    )
    def gk(table_hbm, idx_hbm, out_hbm, idx_v, rows_v, sem):
        wid = lax.axis_index("s") * nc + lax.axis_index("c")

        def body(i, _):
            base = wid * per_w + i * _CH
            pltpu.sync_copy(idx_hbm.at[pl.ds(base, _CH)], idx_v)
            pltpu.async_copy(table_hbm.at[idx_v], rows_v, sem).wait()
            pltpu.sync_copy(rows_v, out_hbm.at[pl.ds(base, _CH)])
            return 0

        lax.fori_loop(0, n_ch, body, 0)

    return gk(table, flat_idx)


def _stats1_kern(v_ref, qT_ref, w1aT_ref, w1xT_ref, s1_ref, s2_ref):
    h1 = jnp.dot(v_ref[...], w1aT_ref[...], preferred_element_type=jnp.float32)
    h1 = h1 - jnp.dot(qT_ref[0], w1xT_ref[...], preferred_element_type=jnp.float32)

    @pl.when(pl.program_id(0) == 0)
    def _():
        s1_ref[...] = jnp.zeros_like(s1_ref)
        s2_ref[...] = jnp.zeros_like(s2_ref)

    s1_ref[...] += jnp.sum(h1, axis=0, keepdims=True)
    s2_ref[...] += jnp.sum(h1 * h1, axis=0, keepdims=True)


def _stats1(v, qT, w1aT, w1xT):
    return pl.pallas_call(
        _stats1_kern,
        grid=(_TOTAL // _ST,),
        in_specs=[
            pl.BlockSpec((_ST, _C), lambda s: (s, 0)),
            pl.BlockSpec((1, _M, 3), lambda s: (s // _K, 0, 0)),
            pl.BlockSpec((_C, 64), lambda s: (0, 0)),
            pl.BlockSpec((3, 64), lambda s: (0, 0)),
        ],
        out_specs=[
            pl.BlockSpec((1, 64), lambda s: (0, 0)),
            pl.BlockSpec((1, 64), lambda s: (0, 0)),
        ],
        out_shape=[
            jax.ShapeDtypeStruct((1, 64), jnp.float32),
            jax.ShapeDtypeStruct((1, 64), jnp.float32),
        ],
    )(v, qT, w1aT, w1xT)


def _stats2_kern(v_ref, qT_ref, w1aT_ref, w1xT_ref, s1_ref, s2_ref,
                 g1_ref, b1_ref, w2T_ref, t1_ref, t2_ref):
    h1 = jnp.dot(v_ref[...], w1aT_ref[...], preferred_element_type=jnp.float32)
    h1 = h1 - jnp.dot(qT_ref[0], w1xT_ref[...], preferred_element_type=jnp.float32)
    mu1 = s1_ref[...] / _TOTAL
    var1 = s2_ref[...] / _TOTAL - mu1 * mu1
    sc1 = g1_ref[...] * lax.rsqrt(var1 + _EPS)
    h1r = jnp.maximum((h1 - mu1) * sc1 + b1_ref[...], 0.0)
    h2 = jnp.dot(h1r, w2T_ref[...], preferred_element_type=jnp.float32)

    @pl.when(pl.program_id(0) == 0)
    def _():
        t1_ref[...] = jnp.zeros_like(t1_ref)
        t2_ref[...] = jnp.zeros_like(t2_ref)

    t1_ref[...] += jnp.sum(h2, axis=0, keepdims=True)
    t2_ref[...] += jnp.sum(h2 * h2, axis=0, keepdims=True)


def _stats2(v, qT, w1aT, w1xT, s1, s2, g1r, b1r, w2T):
    return pl.pallas_call(
        _stats2_kern,
        grid=(_TOTAL // _ST,),
        in_specs=[
            pl.BlockSpec((_ST, _C), lambda s: (s, 0)),
            pl.BlockSpec((1, _M, 3), lambda s: (s // _K, 0, 0)),
            pl.BlockSpec((_C, 64), lambda s: (0, 0)),
            pl.BlockSpec((3, 64), lambda s: (0, 0)),
            pl.BlockSpec((1, 64), lambda s: (0, 0)),
            pl.BlockSpec((1, 64), lambda s: (0, 0)),
            pl.BlockSpec((1, 64), lambda s: (0, 0)),
            pl.BlockSpec((1, 64), lambda s: (0, 0)),
            pl.BlockSpec((64, 128), lambda s: (0, 0)),
        ],
        out_specs=[
            pl.BlockSpec((1, 128), lambda s: (0, 0)),
            pl.BlockSpec((1, 128), lambda s: (0, 0)),
        ],
        out_shape=[
            jax.ShapeDtypeStruct((1, 128), jnp.float32),
            jax.ShapeDtypeStruct((1, 128), jnp.float32),
        ],
    )(v, qT, w1aT, w1xT, s1, s2, g1r, b1r, w2T)


def _final_kern(v_ref, qT_ref, w1aT_ref, w1xT_ref, s1_ref, s2_ref,
                g1_ref, b1_ref, w2T_ref, t1_ref, t2_ref, g2_ref, b2_ref,
                o_ref):
    v2 = v_ref[0].reshape(_K * _MT, _C)
    h1 = jnp.dot(v2, w1aT_ref[...], preferred_element_type=jnp.float32)
    pt = jnp.dot(qT_ref[0], w1xT_ref[...], preferred_element_type=jnp.float32)
    h1 = (h1.reshape(_K, _MT, 64) - pt[None]).reshape(_K * _MT, 64)
    mu1 = s1_ref[...] / _TOTAL
    var1 = s2_ref[...] / _TOTAL - mu1 * mu1
    sc1 = g1_ref[...] * lax.rsqrt(var1 + _EPS)
    h1r = jnp.maximum((h1 - mu1) * sc1 + b1_ref[...], 0.0)
    h2 = jnp.dot(h1r, w2T_ref[...], preferred_element_type=jnp.float32)
    mu2 = t1_ref[...] / _TOTAL
    var2 = t2_ref[...] / _TOTAL - mu2 * mu2
    sc2 = g2_ref[...] * lax.rsqrt(var2 + _EPS)
    h2r = jnp.maximum((h2 - mu2) * sc2 + b2_ref[...], 0.0)
    o_ref[0] = jnp.max(h2r.reshape(_K, _MT, 128), axis=0)


def _final(v4, qT, w1aT, w1xT, s1, s2, g1r, b1r, w2T, t1, t2, g2r, b2r):
    return pl.pallas_call(
        _final_kern,
        grid=(_B, _M // _MT),
        in_specs=[
            pl.BlockSpec((1, _K, _MT, _C), lambda b, t: (b, 0, t, 0)),
            pl.BlockSpec((1, _MT, 3), lambda b, t: (b, t, 0)),
            pl.BlockSpec((_C, 64), lambda b, t: (0, 0)),
            pl.BlockSpec((3, 64), lambda b, t: (0, 0)),
            pl.BlockSpec((1, 64), lambda b, t: (0, 0)),
            pl.BlockSpec((1, 64), lambda b, t: (0, 0)),
            pl.BlockSpec((1, 64), lambda b, t: (0, 0)),
            pl.BlockSpec((1, 64), lambda b, t: (0, 0)),
            pl.BlockSpec((64, 128), lambda b, t: (0, 0)),
            pl.BlockSpec((1, 128), lambda b, t: (0, 0)),
            pl.BlockSpec((1, 128), lambda b, t: (0, 0)),
            pl.BlockSpec((1, 128), lambda b, t: (0, 0)),
            pl.BlockSpec((1, 128), lambda b, t: (0, 0)),
        ],
        out_specs=pl.BlockSpec((1, _MT, 128), lambda b, t: (b, t, 0)),
        out_shape=jax.ShapeDtypeStruct((_B, _M, 128), jnp.float32),
    )(v4, qT, w1aT, w1xT, s1, s2, g1r, b1r, w2T, t1, t2, g2r, b2r)


def kernel(x, W1, g1, b1, W2, g2, b2):
    x3 = x[:, :, :, 0]
    pts = x3[:, 0:3, :]
    qT = jnp.transpose(x3[:, 0:3, ::_DS], (0, 2, 1))

    idx = _knn(pts, qT)

    table = jnp.transpose(x3, (0, 2, 1)).reshape(_B * _N, _C)
    flat_idx = (idx + (jnp.arange(_B, dtype=jnp.int32) * _N)[:, None, None]
                ).reshape(-1)
    v = _gather_sc(table, flat_idx)

    w1a = jnp.concatenate(
        [W1[:, 0:6], jnp.zeros((64, 1), jnp.float32), W1[:, 6:15]], axis=1)
    w1aT = w1a.T
    w1xT = W1[:, 0:3].T
    g1r, b1r = g1.reshape(1, 64), b1.reshape(1, 64)
    g2r, b2r = g2.reshape(1, 128), b2.reshape(1, 128)
    w2T = W2.T

    s1, s2 = _stats1(v, qT, w1aT, w1xT)
    t1, t2 = _stats2(v, qT, w1aT, w1xT, s1, s2, g1r, b1r, w2T)
    o = _final(v.reshape(_B, _K, _M, _C), qT, w1aT, w1xT,
               s1, s2, g1r, b1r, w2T, t1, t2, g2r, b2r)

    pd = x[:, 0:_XYZN, ::_DS, :]
    return jnp.concatenate(
        [pd, jnp.transpose(o, (0, 2, 1))[..., None]], axis=1)


# speedup vs baseline: 6.6337x; 6.6337x over previous
import functools

import jax
import jax.numpy as jnp
from jax import lax
from jax.experimental import pallas as pl
from jax.experimental.pallas import tpu as pltpu
from jax.experimental.pallas import tpu_sc as plsc

_B, _C, _N = 2, 16, 8192
_DS = 4
_M = _N // _DS
_K = 32
_XYZN = 7
_EPS = 1e-5
_QT = 128
_ST = 2048
_MT = 256
_TOTAL = _B * _K * _M


def _knn_kern(pts_ref, q_ref, out_ref, d_ref):
    px = pts_ref[0, 0:1, :]
    py = pts_ref[0, 1:2, :]
    pz = pts_ref[0, 2:3, :]
    qx = q_ref[0, :, 0:1]
    qy = q_ref[0, :, 1:2]
    qz = q_ref[0, :, 2:3]
    d_ref[...] = (qx - px) ** 2 + (qy - py) ** 2 + (qz - pz) ** 2
    iota = lax.broadcasted_iota(jnp.int32, (_QT, _N), 1)

    def body(k, _):
        d = d_ref[...]
        mn = jnp.min(d, axis=1, keepdims=True)
        am = jnp.min(jnp.where(d == mn, iota, _N), axis=1)
        out_ref[0, pl.ds(k, 1), :] = am[None, :]
        d_ref[...] = jnp.where(iota == am[:, None], jnp.inf, d)
        return 0

    lax.fori_loop(0, _K, body, 0)


def _knn(pts, qT):
    return pl.pallas_call(
        _knn_kern,
        grid=(_B, _M // _QT),
        in_specs=[
            pl.BlockSpec((1, 3, _N), lambda b, t: (b, 0, 0)),
            pl.BlockSpec((1, _QT, 3), lambda b, t: (b, t, 0)),
        ],
        out_specs=pl.BlockSpec((1, _K, _QT), lambda b, t: (b, 0, t)),
        out_shape=jax.ShapeDtypeStruct((_B, _K, _M), jnp.int32),
        scratch_shapes=[pltpu.VMEM((_QT, _N), jnp.float32)],
    )(pts, qT)


_NW = 32
_CH = 128


def _gather_sc(table, flat_idx):
    per_w = _TOTAL // _NW
    n_ch = per_w // _CH
    info = plsc.get_sparse_core_info()
    nc = info.num_cores
    mesh = plsc.VectorSubcoreMesh(core_axis_name="c", subcore_axis_name="s")

    @functools.partial(
        pl.kernel,
        mesh=mesh,
        compiler_params=pltpu.CompilerParams(use_tc_tiling_on_sc=False),
        out_type=jax.ShapeDtypeStruct((_TOTAL, _C), jnp.float32),
        scratch_types=[
            pltpu.VMEM((_CH,), jnp.int32),
            pltpu.VMEM((_CH, _C), jnp.float32),
            pltpu.SemaphoreType.DMA,
        ],
    )
    def gk(table_hbm, idx_hbm, out_hbm, idx_v, rows_v, sem):
        wid = lax.axis_index("s") * nc + lax.axis_index("c")

        def body(i, _):
            base = wid * per_w + i * _CH
            pltpu.sync_copy(idx_hbm.at[pl.ds(base, _CH)], idx_v)
            pltpu.async_copy(table_hbm.at[idx_v], rows_v, sem).wait()
            pltpu.sync_copy(rows_v, out_hbm.at[pl.ds(base, _CH)])
            return 0

        lax.fori_loop(0, n_ch, body, 0)

    return gk(table, flat_idx)


def _stats1_kern(v_ref, qT_ref, w1aT_ref, w1xT_ref, s1_ref, s2_ref):
    h1 = jnp.dot(v_ref[...], w1aT_ref[...], preferred_element_type=jnp.float32)
    h1 = h1 - jnp.dot(qT_ref[0], w1xT_ref[...], preferred_element_type=jnp.float32)

    @pl.when(pl.program_id(0) == 0)
    def _():
        s1_ref[...] = jnp.zeros_like(s1_ref)
        s2_ref[...] = jnp.zeros_like(s2_ref)

    s1_ref[...] += jnp.sum(h1, axis=0, keepdims=True)
    s2_ref[...] += jnp.sum(h1 * h1, axis=0, keepdims=True)


def _stats1(v, qT, w1aT, w1xT):
    return pl.pallas_call(
        _stats1_kern,
        grid=(_TOTAL // _ST,),
        in_specs=[
            pl.BlockSpec((_ST, _C), lambda s: (s, 0)),
            pl.BlockSpec((1, _M, 3), lambda s: (s // _K, 0, 0)),
            pl.BlockSpec((_C, 64), lambda s: (0, 0)),
            pl.BlockSpec((3, 64), lambda s: (0, 0)),
        ],
        out_specs=[
            pl.BlockSpec((1, 64), lambda s: (0, 0)),
            pl.BlockSpec((1, 64), lambda s: (0, 0)),
        ],
        out_shape=[
            jax.ShapeDtypeStruct((1, 64), jnp.float32),
            jax.ShapeDtypeStruct((1, 64), jnp.float32),
        ],
    )(v, qT, w1aT, w1xT)


def _stats2_kern(v_ref, qT_ref, w1aT_ref, w1xT_ref, s1_ref, s2_ref,
                 g1_ref, b1_ref, w2T_ref, t1_ref, t2_ref):
    h1 = jnp.dot(v_ref[...], w1aT_ref[...], preferred_element_type=jnp.float32)
    h1 = h1 - jnp.dot(qT_ref[0], w1xT_ref[...], preferred_element_type=jnp.float32)
    mu1 = s1_ref[...] / _TOTAL
    var1 = s2_ref[...] / _TOTAL - mu1 * mu1
    sc1 = g1_ref[...] * lax.rsqrt(var1 + _EPS)
    h1r = jnp.maximum((h1 - mu1) * sc1 + b1_ref[...], 0.0)
    h2 = jnp.dot(h1r, w2T_ref[...], preferred_element_type=jnp.float32)

    @pl.when(pl.program_id(0) == 0)
    def _():
        t1_ref[...] = jnp.zeros_like(t1_ref)
        t2_ref[...] = jnp.zeros_like(t2_ref)

    t1_ref[...] += jnp.sum(h2, axis=0, keepdims=True)
    t2_ref[...] += jnp.sum(h2 * h2, axis=0, keepdims=True)


def _stats2(v, qT, w1aT, w1xT, s1, s2, g1r, b1r, w2T):
    return pl.pallas_call(
        _stats2_kern,
        grid=(_TOTAL // _ST,),
        in_specs=[
            pl.BlockSpec((_ST, _C), lambda s: (s, 0)),
            pl.BlockSpec((1, _M, 3), lambda s: (s // _K, 0, 0)),
            pl.BlockSpec((_C, 64), lambda s: (0, 0)),
            pl.BlockSpec((3, 64), lambda s: (0, 0)),
            pl.BlockSpec((1, 64), lambda s: (0, 0)),
            pl.BlockSpec((1, 64), lambda s: (0, 0)),
            pl.BlockSpec((1, 64), lambda s: (0, 0)),
            pl.BlockSpec((1, 64), lambda s: (0, 0)),
            pl.BlockSpec((64, 128), lambda s: (0, 0)),
        ],
        out_specs=[
            pl.BlockSpec((1, 128), lambda s: (0, 0)),
            pl.BlockSpec((1, 128), lambda s: (0, 0)),
        ],
        out_shape=[
            jax.ShapeDtypeStruct((1, 128), jnp.float32),
            jax.ShapeDtypeStruct((1, 128), jnp.float32),
        ],
    )(v, qT, w1aT, w1xT, s1, s2, g1r, b1r, w2T)


def _final_kern(v_ref, qT_ref, w1aT_ref, w1xT_ref, s1_ref, s2_ref,
                g1_ref, b1_ref, w2T_ref, t1_ref, t2_ref, g2_ref, b2_ref,
                o_ref):
    v2 = v_ref[0].reshape(_K * _MT, _C)
    h1 = jnp.dot(v2, w1aT_ref[...], preferred_element_type=jnp.float32)
    pt = jnp.dot(qT_ref[0], w1xT_ref[...], preferred_element_type=jnp.float32)
    h1 = (h1.reshape(_K, _MT, 64) - pt[None]).reshape(_K * _MT, 64)
    mu1 = s1_ref[...] / _TOTAL
    var1 = s2_ref[...] / _TOTAL - mu1 * mu1
    sc1 = g1_ref[...] * lax.rsqrt(var1 + _EPS)
    h1r = jnp.maximum((h1 - mu1) * sc1 + b1_ref[...], 0.0)
    h2 = jnp.dot(h1r, w2T_ref[...], preferred_element_type=jnp.float32)
    mu2 = t1_ref[...] / _TOTAL
    var2 = t2_ref[...] / _TOTAL - mu2 * mu2
    sc2 = g2_ref[...] * lax.rsqrt(var2 + _EPS)
    h2r = jnp.maximum((h2 - mu2) * sc2 + b2_ref[...], 0.0)
    o_ref[0] = jnp.max(h2r.reshape(_K, _MT, 128), axis=0)


def _final(v4, qT, w1aT, w1xT, s1, s2, g1r, b1r, w2T, t1, t2, g2r, b2r):
    return pl.pallas_call(
        _final_kern,
        grid=(_B, _M // _MT),
        in_specs=[
            pl.BlockSpec((1, _K, _MT, _C), lambda b, t: (b, 0, t, 0)),
            pl.BlockSpec((1, _MT, 3), lambda b, t: (b, t, 0)),
            pl.BlockSpec((_C, 64), lambda b, t: (0, 0)),
            pl.BlockSpec((3, 64), lambda b, t: (0, 0)),
            pl.BlockSpec((1, 64), lambda b, t: (0, 0)),
            pl.BlockSpec((1, 64), lambda b, t: (0, 0)),
            pl.BlockSpec((1, 64), lambda b, t: (0, 0)),
            pl.BlockSpec((1, 64), lambda b, t: (0, 0)),
            pl.BlockSpec((64, 128), lambda b, t: (0, 0)),
            pl.BlockSpec((1, 128), lambda b, t: (0, 0)),
            pl.BlockSpec((1, 128), lambda b, t: (0, 0)),
            pl.BlockSpec((1, 128), lambda b, t: (0, 0)),
            pl.BlockSpec((1, 128), lambda b, t: (0, 0)),
        ],
        out_specs=pl.BlockSpec((1, _MT, 128), lambda b, t: (b, t, 0)),
        out_shape=jax.ShapeDtypeStruct((_B, _M, 128), jnp.float32),
    )(v4, qT, w1aT, w1xT, s1, s2, g1r, b1r, w2T, t1, t2, g2r, b2r)


def kernel(x, W1, g1, b1, W2, g2, b2):
    x3 = x[:, :, :, 0]
    pts = x3[:, 0:3, :]
    qT = jnp.transpose(x3[:, 0:3, ::_DS], (0, 2, 1))

    idx = _knn(pts, qT)

    table = jnp.transpose(x3, (0, 2, 1)).reshape(_B * _N, _C)
    flat_idx = (idx + (jnp.arange(_B, dtype=jnp.int32) * _N)[:, None, None]
                ).reshape(-1)
    v = _gather_sc(table, flat_idx)

    w1a = jnp.concatenate(
        [W1[:, 0:6], jnp.zeros((64, 1), jnp.float32), W1[:, 6:15]], axis=1)
    w1aT = w1a.T
    w1xT = W1[:, 0:3].T
    g1r, b1r = g1.reshape(1, 64), b1.reshape(1, 64)
    g2r, b2r = g2.reshape(1, 128), b2.reshape(1, 128)
    w2T = W2.T

    s1, s2 = _stats1(v, qT, w1aT, w1xT)
    t1, t2 = _stats2(v, qT, w1aT, w1xT, s1, s2, g1r, b1r, w2T)
    o = _final(v.reshape(_B, _K, _M, _C), qT, w1aT, w1xT,
               s1, s2, g1r, b1r, w2T, t1, t2, g2r, b2r)

    pd = x[:, 0:_XYZN, ::_DS, :]
    return jnp.concatenate(
        [pd, jnp.transpose(o, (0, 2, 1))[..., None]], axis=1)
